# baseline (speedup 1.0000x reference)
import functools
import math

import jax
import jax.numpy as jnp
from jax import lax
from jax.experimental import pallas as pl
from jax.experimental.pallas import tpu as pltpu
from jax.experimental.pallas import tpu_sc as plsc

VOCAB = 1000000
EMBED = 64
WINDOW = 200
BATCH = 4096

NUM_CORES = 2
NUM_SUBCORES = 16
NUM_WORKERS = NUM_CORES * NUM_SUBCORES

PADROW = 2 * EMBED
ROWS_PER_WORKER = BATCH // NUM_WORKERS
HALF = WINDOW // 2
SCALE = math.sqrt(EMBED)


def _tc_row_image(tT):
    CB = 512

    def body(in_ref, out_ref):
        eye = jnp.eye(EMBED, dtype=jnp.float32)
        t = lax.dot_general(in_ref[...], eye, (((0,), (0,)), ((), ())),
                            preferred_element_type=jnp.float32)
        out_ref[...] = jnp.concatenate([t, t], axis=1)

    return pl.pallas_call(
        body,
        grid=((VOCAB + CB - 1) // CB,),
        in_specs=[pl.BlockSpec((EMBED, CB), lambda j: (0, j))],
        out_specs=pl.BlockSpec((CB, PADROW), lambda j: (j, 0)),
        out_shape=jax.ShapeDtypeStruct((VOCAB, PADROW), jnp.float32),
    )(tT)


def _sc_embed(x2, t2, pos_enc):
    mesh = plsc.VectorSubcoreMesh(core_axis_name="c", subcore_axis_name="s")

    @functools.partial(
        pl.kernel,
        mesh=mesh,
        compiler_params=pltpu.CompilerParams(use_tc_tiling_on_sc=False,
                                             needs_layout_passes=False),
        out_type=jax.ShapeDtypeStruct((BATCH, WINDOW, PADROW), jnp.float32),
        scratch_types=[
            pltpu.VMEM((2 * ROWS_PER_WORKER, HALF), jnp.int32),
            pltpu.VMEM((WINDOW, EMBED), jnp.float32),
            pltpu.VMEM((2, WINDOW, PADROW), jnp.float32),
            pltpu.SemaphoreType.DMA,
            pltpu.SemaphoreType.DMA,
        ],
    )
    def k(x_hbm, t2_hbm, pos_hbm, out_hbm, idx_all, pos_v, gbuf,
          sem_g, sem_s):
        wid = lax.axis_index("s") * NUM_CORES + lax.axis_index("c")
        r0 = wid * ROWS_PER_WORKER

        pltpu.sync_copy(pos_hbm, pos_v)
        pltpu.sync_copy(x_hbm.at[pl.ds(2 * r0, 2 * ROWS_PER_WORKER)],
                        idx_all)

        def start_gather(u, p):
            pltpu.async_copy(t2_hbm.at[idx_all.at[2 * u]],
                             gbuf.at[p, pl.ds(0, HALF)], sem_g)
            pltpu.async_copy(t2_hbm.at[idx_all.at[2 * u + 1]],
                             gbuf.at[p, pl.ds(HALF, HALF)], sem_g)

        def wait_gather(p):
            pltpu.make_async_copy(t2_hbm.at[pl.ds(0, WINDOW)], gbuf.at[p],
                                  sem_g).wait()

        def start_scatter(u, p):
            pltpu.async_copy(gbuf.at[p], out_hbm.at[r0 + u], sem_s)

        def wait_scatter(p):
            pltpu.make_async_copy(gbuf.at[p], out_hbm.at[0], sem_s).wait()

        def compute(p):
            def rows(r2, _):
                for dr in range(2):
                    r = r2 * 2 + dr
                    for q in range(EMBED // 16):
                        v = gbuf[p, r, pl.ds(q * 16, 16)]
                        po = pos_v[r, pl.ds(q * 16, 16)]
                        gbuf[p, r, pl.ds(q * 16, 16)] = v * SCALE + po
                return 0

            lax.fori_loop(0, WINDOW // 2, rows, 0)

        start_gather(0, 0)
        wait_gather(0)
        start_gather(1, 1)
        compute(0)
        start_scatter(0, 0)
        wait_gather(1)
        wait_scatter(0)
        start_gather(2, 0)
        compute(1)
        start_scatter(1, 1)

        def body(t, _):
            for dr in range(2):
                u = t * 2 + dr
                p = dr
                wait_gather(p)
                wait_scatter(1 - p)
                start_gather(u + 1, 1 - p)
                compute(p)
                start_scatter(u, p)
            return 0

        lax.fori_loop(1, ROWS_PER_WORKER // 2 - 1, body, 0)

        wait_gather(0)
        wait_scatter(1)
        start_gather(ROWS_PER_WORKER - 1, 1)
        compute(0)
        start_scatter(ROWS_PER_WORKER - 2, 0)
        wait_gather(1)
        compute(1)
        start_scatter(ROWS_PER_WORKER - 1, 1)

        wait_scatter(0)
        wait_scatter(1)

    return k(x2, t2, pos_enc)


def kernel(x, table, pos_enc):
    x2 = x.astype(jnp.int32).reshape(2 * BATCH, HALF)
    t2 = _tc_row_image(jnp.transpose(table))
    out = _sc_embed(x2, t2, pos_enc)
    return out[:, :, :EMBED]


# speedup vs baseline: 2.7050x; 2.7050x over previous
import functools
import math

import jax
import jax.numpy as jnp
from jax import lax
from jax.experimental import pallas as pl
from jax.experimental.pallas import tpu as pltpu
from jax.experimental.pallas import tpu_sc as plsc

VOCAB = 1000000
EMBED = 64
WINDOW = 200
BATCH = 4096

NUM_CORES = 2
NUM_SUBCORES = 16
NUM_WORKERS = NUM_CORES * NUM_SUBCORES

PADROW = 2 * EMBED
ROWS_PER_WORKER = BATCH // NUM_WORKERS
HALF = WINDOW // 2
SCALE = math.sqrt(EMBED)


def _tc_row_image(tT):
    CB = 4096

    def body(in_ref, out_ref):
        eye = jnp.eye(EMBED, dtype=jnp.float32)
        t = lax.dot_general(in_ref[...], eye, (((0,), (0,)), ((), ())),
                            preferred_element_type=jnp.float32)
        out_ref[...] = jnp.concatenate([t, t], axis=1)

    return pl.pallas_call(
        body,
        grid=((VOCAB + CB - 1) // CB,),
        in_specs=[pl.BlockSpec((EMBED, CB), lambda j: (0, j))],
        out_specs=pl.BlockSpec((CB, PADROW), lambda j: (j, 0)),
        out_shape=jax.ShapeDtypeStruct((VOCAB, PADROW), jnp.float32),
    )(tT)


def _sc_embed(x2, t2b, pos_enc):
    mesh = plsc.VectorSubcoreMesh(core_axis_name="c", subcore_axis_name="s")

    @functools.partial(
        pl.kernel,
        mesh=mesh,
        compiler_params=pltpu.CompilerParams(use_tc_tiling_on_sc=False,
                                             needs_layout_passes=False),
        out_type=jax.ShapeDtypeStruct((BATCH, WINDOW, PADROW), jnp.float32),
        scratch_types=[
            pltpu.VMEM((2 * ROWS_PER_WORKER, HALF), jnp.int32),
            pltpu.VMEM((WINDOW, EMBED), jnp.float32),
            pltpu.VMEM((2, WINDOW, EMBED), jnp.float32),
            pltpu.SemaphoreType.DMA,
            pltpu.SemaphoreType.DMA,
        ],
    )
    def k(x_hbm, t2_hbm, pos_hbm, out_hbm, idx_all, pos_v, gbuf,
          sem_g, sem_s):
        wid = lax.axis_index("s") * NUM_CORES + lax.axis_index("c")
        r0 = wid * ROWS_PER_WORKER

        pltpu.sync_copy(pos_hbm, pos_v)
        pltpu.sync_copy(x_hbm.at[pl.ds(2 * r0, 2 * ROWS_PER_WORKER)],
                        idx_all)

        def start_gather(u, p):
            pltpu.async_copy(t2_hbm.at[idx_all.at[2 * u]],
                             gbuf.at[p, pl.ds(0, HALF)], sem_g)
            pltpu.async_copy(t2_hbm.at[idx_all.at[2 * u + 1]],
                             gbuf.at[p, pl.ds(HALF, HALF)], sem_g)

        def wait_gather(p):
            pltpu.make_async_copy(t2_hbm.at[pl.ds(0, WINDOW)], gbuf.at[p],
                                  sem_g).wait()

        def start_scatter(u, p):
            pltpu.async_copy(
                gbuf.at[p],
                out_hbm.at[r0 + u, pl.ds(0, WINDOW), pl.ds(0, EMBED)], sem_s)

        def wait_scatter(p):
            pltpu.make_async_copy(
                gbuf.at[p],
                out_hbm.at[0, pl.ds(0, WINDOW), pl.ds(0, EMBED)],
                sem_s).wait()

        def compute(p):
            def rows(r2, _):
                for dr in range(2):
                    r = r2 * 2 + dr
                    for q in range(EMBED // 16):
                        v = gbuf[p, r, pl.ds(q * 16, 16)]
                        po = pos_v[r, pl.ds(q * 16, 16)]
                        gbuf[p, r, pl.ds(q * 16, 16)] = v * SCALE + po
                return 0

            lax.fori_loop(0, WINDOW // 2, rows, 0)

        start_gather(0, 0)
        wait_gather(0)
        start_gather(1, 1)
        compute(0)
        start_scatter(0, 0)
        wait_gather(1)
        wait_scatter(0)
        start_gather(2, 0)
        compute(1)
        start_scatter(1, 1)

        def body(t, _):
            for dr in range(2):
                u = t * 2 + dr
                p = dr
                wait_gather(p)
                wait_scatter(1 - p)
                start_gather(u + 1, 1 - p)
                compute(p)
                start_scatter(u, p)
            return 0

        lax.fori_loop(1, ROWS_PER_WORKER // 2 - 1, body, 0)

        wait_gather(0)
        wait_scatter(1)
        start_gather(ROWS_PER_WORKER - 1, 1)
        compute(0)
        start_scatter(ROWS_PER_WORKER - 2, 0)
        wait_gather(1)
        compute(1)
        start_scatter(ROWS_PER_WORKER - 1, 1)

        wait_scatter(0)
        wait_scatter(1)

    return k(x2, t2b, pos_enc)


def kernel(x, table, pos_enc):
    x2 = (x.astype(jnp.int32) * 2).reshape(2 * BATCH, HALF)
    t2 = _tc_row_image(jnp.transpose(table))
    t2b = t2.reshape(2 * VOCAB, EMBED)
    out = _sc_embed(x2, t2b, pos_enc)
    return out[:, :, :EMBED]


# speedup vs baseline: 2.9626x; 1.0952x over previous
import functools
import math

import jax
import jax.numpy as jnp
from jax import lax
from jax.experimental import pallas as pl
from jax.experimental.pallas import tpu as pltpu
from jax.experimental.pallas import tpu_sc as plsc

VOCAB = 1000000
EMBED = 64
WINDOW = 200
BATCH = 4096

NUM_CORES = 2
NUM_SUBCORES = 16
NUM_WORKERS = NUM_CORES * NUM_SUBCORES

PADROW = 2 * EMBED
ROWS_PER_WORKER = BATCH // NUM_WORKERS
HALF = WINDOW // 2
SCALE = math.sqrt(EMBED)


def _tc_row_image(tT):
    CB = 8192

    def body(in_ref, out_ref):
        eye = jnp.eye(EMBED, dtype=jnp.float32)
        t = lax.dot_general(in_ref[...], eye, (((0,), (0,)), ((), ())),
                            preferred_element_type=jnp.float32)
        out_ref[...] = jnp.concatenate([t, t], axis=1)

    return pl.pallas_call(
        body,
        grid=((VOCAB + CB - 1) // CB,),
        in_specs=[pl.BlockSpec((EMBED, CB), lambda j: (0, j))],
        out_specs=pl.BlockSpec((CB, PADROW), lambda j: (j, 0)),
        out_shape=jax.ShapeDtypeStruct((VOCAB, PADROW), jnp.float32),
    )(tT)


def _sc_embed(x2, t2b, pos_enc):
    mesh = plsc.VectorSubcoreMesh(core_axis_name="c", subcore_axis_name="s")

    @functools.partial(
        pl.kernel,
        mesh=mesh,
        compiler_params=pltpu.CompilerParams(use_tc_tiling_on_sc=False,
                                             needs_layout_passes=False),
        out_type=jax.ShapeDtypeStruct((BATCH, WINDOW, PADROW), jnp.float32),
        scratch_types=[
            pltpu.VMEM((2 * ROWS_PER_WORKER, HALF), jnp.int32),
            pltpu.VMEM((WINDOW, EMBED), jnp.float32),
            pltpu.VMEM((2, WINDOW, EMBED), jnp.float32),
            pltpu.SemaphoreType.DMA,
            pltpu.SemaphoreType.DMA,
        ],
    )
    def k(x_hbm, t2_hbm, pos_hbm, out_hbm, idx_all, pos_v, gbuf,
          sem_g, sem_s):
        wid = lax.axis_index("s") * NUM_CORES + lax.axis_index("c")
        r0 = wid * ROWS_PER_WORKER

        pltpu.sync_copy(pos_hbm, pos_v)
        pltpu.sync_copy(x_hbm.at[pl.ds(2 * r0, 2 * ROWS_PER_WORKER)],
                        idx_all)

        def start_gather(u, p):
            pltpu.async_copy(t2_hbm.at[idx_all.at[2 * u]],
                             gbuf.at[p, pl.ds(0, HALF)], sem_g)
            pltpu.async_copy(t2_hbm.at[idx_all.at[2 * u + 1]],
                             gbuf.at[p, pl.ds(HALF, HALF)], sem_g)

        def wait_gather(p):
            pltpu.make_async_copy(t2_hbm.at[pl.ds(0, WINDOW)], gbuf.at[p],
                                  sem_g).wait()

        def start_scatter(u, p):
            pltpu.async_copy(
                gbuf.at[p],
                out_hbm.at[r0 + u, pl.ds(0, WINDOW), pl.ds(0, EMBED)], sem_s)

        def wait_scatter(p):
            pltpu.make_async_copy(
                gbuf.at[p],
                out_hbm.at[0, pl.ds(0, WINDOW), pl.ds(0, EMBED)],
                sem_s).wait()

        def compute(p):
            def rows(r2, _):
                for dr in range(2):
                    r = r2 * 2 + dr
                    for q in range(EMBED // 16):
                        v = gbuf[p, r, pl.ds(q * 16, 16)]
                        po = pos_v[r, pl.ds(q * 16, 16)]
                        gbuf[p, r, pl.ds(q * 16, 16)] = v * SCALE + po
                return 0

            lax.fori_loop(0, WINDOW // 2, rows, 0)

        start_gather(0, 0)
        wait_gather(0)
        start_gather(1, 1)
        compute(0)
        start_scatter(0, 0)
        wait_gather(1)
        wait_scatter(0)
        start_gather(2, 0)
        compute(1)
        start_scatter(1, 1)

        def body(t, _):
            for dr in range(2):
                u = t * 2 + dr
                p = dr
                wait_gather(p)
                wait_scatter(1 - p)
                start_gather(u + 1, 1 - p)
                compute(p)
                start_scatter(u, p)
            return 0

        lax.fori_loop(1, ROWS_PER_WORKER // 2 - 1, body, 0)

        wait_gather(0)
        wait_scatter(1)
        start_gather(ROWS_PER_WORKER - 1, 1)
        compute(0)
        start_scatter(ROWS_PER_WORKER - 2, 0)
        wait_gather(1)
        compute(1)
        start_scatter(ROWS_PER_WORKER - 1, 1)

        wait_scatter(0)
        wait_scatter(1)

    return k(x2, t2b, pos_enc)


def kernel(x, table, pos_enc):
    x2 = (x.astype(jnp.int32) * 2).reshape(2 * BATCH, HALF)
    t2 = _tc_row_image(jnp.transpose(table))
    t2b = t2.reshape(2 * VOCAB, EMBED)
    out = _sc_embed(x2, t2b, pos_enc)
    return out[:, :, :EMBED]


# speedup vs baseline: 3.1025x; 1.0472x over previous
import functools
import math

import jax
import jax.numpy as jnp
from jax import lax
from jax.experimental import pallas as pl
from jax.experimental.pallas import tpu as pltpu
from jax.experimental.pallas import tpu_sc as plsc

VOCAB = 1000000
EMBED = 64
WINDOW = 200
BATCH = 4096

NUM_CORES = 2
NUM_SUBCORES = 16
NUM_WORKERS = NUM_CORES * NUM_SUBCORES

PADROW = 2 * EMBED
ROWS_PER_WORKER = BATCH // NUM_WORKERS
HALF = WINDOW // 2
SCALE = math.sqrt(EMBED)


def _tc_row_image(tT):
    CB = 16384

    def body(in_ref, out_ref):
        eye = jnp.eye(EMBED, dtype=jnp.float32)
        t = lax.dot_general(in_ref[...], eye, (((0,), (0,)), ((), ())),
                            preferred_element_type=jnp.float32)
        out_ref[...] = jnp.concatenate([t, t], axis=1)

    return pl.pallas_call(
        body,
        grid=((VOCAB + CB - 1) // CB,),
        in_specs=[pl.BlockSpec((EMBED, CB), lambda j: (0, j))],
        out_specs=pl.BlockSpec((CB, PADROW), lambda j: (j, 0)),
        out_shape=jax.ShapeDtypeStruct((VOCAB, PADROW), jnp.float32),
    )(tT)


def _sc_embed(x2, t2b, pos_enc):
    mesh = plsc.VectorSubcoreMesh(core_axis_name="c", subcore_axis_name="s")

    @functools.partial(
        pl.kernel,
        mesh=mesh,
        compiler_params=pltpu.CompilerParams(use_tc_tiling_on_sc=False,
                                             needs_layout_passes=False),
        out_type=jax.ShapeDtypeStruct((BATCH, WINDOW, PADROW), jnp.float32),
        scratch_types=[
            pltpu.VMEM((2 * ROWS_PER_WORKER, HALF), jnp.int32),
            pltpu.VMEM((WINDOW, EMBED), jnp.float32),
            pltpu.VMEM((2, WINDOW, EMBED), jnp.float32),
            pltpu.SemaphoreType.DMA,
            pltpu.SemaphoreType.DMA,
        ],
    )
    def k(x_hbm, t2_hbm, pos_hbm, out_hbm, idx_all, pos_v, gbuf,
          sem_g, sem_s):
        wid = lax.axis_index("s") * NUM_CORES + lax.axis_index("c")
        r0 = wid * ROWS_PER_WORKER

        pltpu.sync_copy(pos_hbm, pos_v)
        pltpu.sync_copy(x_hbm.at[pl.ds(2 * r0, 2 * ROWS_PER_WORKER)],
                        idx_all)

        def start_gather(u, p):
            pltpu.async_copy(t2_hbm.at[idx_all.at[2 * u]],
                             gbuf.at[p, pl.ds(0, HALF)], sem_g)
            pltpu.async_copy(t2_hbm.at[idx_all.at[2 * u + 1]],
                             gbuf.at[p, pl.ds(HALF, HALF)], sem_g)

        def wait_gather(p):
            pltpu.make_async_copy(t2_hbm.at[pl.ds(0, WINDOW)], gbuf.at[p],
                                  sem_g).wait()

        def start_scatter(u, p):
            pltpu.async_copy(
                gbuf.at[p],
                out_hbm.at[r0 + u, pl.ds(0, WINDOW), pl.ds(0, EMBED)], sem_s)

        def wait_scatter(p):
            pltpu.make_async_copy(
                gbuf.at[p],
                out_hbm.at[0, pl.ds(0, WINDOW), pl.ds(0, EMBED)],
                sem_s).wait()

        def compute(p):
            def rows(r2, _):
                for dr in range(2):
                    r = r2 * 2 + dr
                    for q in range(EMBED // 16):
                        v = gbuf[p, r, pl.ds(q * 16, 16)]
                        po = pos_v[r, pl.ds(q * 16, 16)]
                        gbuf[p, r, pl.ds(q * 16, 16)] = v * SCALE + po
                return 0

            lax.fori_loop(0, WINDOW // 2, rows, 0)

        start_gather(0, 0)
        wait_gather(0)
        start_gather(1, 1)
        compute(0)
        start_scatter(0, 0)
        wait_gather(1)
        wait_scatter(0)
        start_gather(2, 0)
        compute(1)
        start_scatter(1, 1)

        def body(t, _):
            for dr in range(2):
                u = t * 2 + dr
                p = dr
                wait_gather(p)
                wait_scatter(1 - p)
                start_gather(u + 1, 1 - p)
                compute(p)
                start_scatter(u, p)
            return 0

        lax.fori_loop(1, ROWS_PER_WORKER // 2 - 1, body, 0)

        wait_gather(0)
        wait_scatter(1)
        start_gather(ROWS_PER_WORKER - 1, 1)
        compute(0)
        start_scatter(ROWS_PER_WORKER - 2, 0)
        wait_gather(1)
        compute(1)
        start_scatter(ROWS_PER_WORKER - 1, 1)

        wait_scatter(0)
        wait_scatter(1)

    return k(x2, t2b, pos_enc)


def kernel(x, table, pos_enc):
    x2 = (x.astype(jnp.int32) * 2).reshape(2 * BATCH, HALF)
    t2 = _tc_row_image(jnp.transpose(table))
    t2b = t2.reshape(2 * VOCAB, EMBED)
    out = _sc_embed(x2, t2b, pos_enc)
    return out[:, :, :EMBED]
